# baseline (speedup 1.0000x reference)
import functools

import jax
import jax.numpy as jnp
from jax import lax
from jax.experimental import pallas as pl
from jax.experimental.pallas import tpu as pltpu
from jax.experimental.pallas import tpu_sc as plsc

N_ROWS = 16384
HIDDEN = 1024
TABLE_ROWS = 1000

SPLIT = 12288

BLOCK_ROWS = 2048
SUB = 16


def _tc_body(idx_smem, feat_ref, idx_vec_ref, table_ref, out_ref):
    j = pl.program_id(0)
    block_base = j * BLOCK_ROWS
    for k in range(BLOCK_ROWS // SUB):
        base = block_base + k * SUB
        r0 = idx_smem[base]
        r1 = idx_smem[base + SUB - 1]
        a = table_ref[pl.ds(r0, 1), :]
        b = table_ref[pl.ds(r1, 1), :]
        idx_v = idx_vec_ref[pl.ds(k * SUB, SUB), :]
        mask = idx_v == r0
        sl = pl.ds(k * SUB, SUB)
        out_ref[sl, :] = feat_ref[sl, :] + jnp.where(mask, a, b)


def _tc_part(features, idx, temporal_embedding):
    grid_spec = pltpu.PrefetchScalarGridSpec(
        num_scalar_prefetch=1,
        grid=(SPLIT // BLOCK_ROWS,),
        in_specs=[
            pl.BlockSpec((BLOCK_ROWS, HIDDEN), lambda i, s: (i, 0)),
            pl.BlockSpec((BLOCK_ROWS, 1), lambda i, s: (i, 0)),
            pl.BlockSpec((TABLE_ROWS, HIDDEN), lambda i, s: (0, 0)),
        ],
        out_specs=pl.BlockSpec((BLOCK_ROWS, HIDDEN), lambda i, s: (i, 0)),
    )
    return pl.pallas_call(
        _tc_body,
        grid_spec=grid_spec,
        out_shape=jax.ShapeDtypeStruct((SPLIT, HIDDEN), features.dtype),
    )(idx, features, idx.reshape(N_ROWS, 1), temporal_embedding)


NC = 2
NS = 16
NW = NC * NS
SC_ROWS = N_ROWS - SPLIT
RPW = SC_ROWS // NW
CHUNK = 16
NBUF = 2
ROUNDS = RPW // CHUNK
GROUPS = ROUNDS // NBUF
LANES = 16


def _sc_body(feat_hbm, idx_hbm, table_hbm, out_hbm,
             idx_v, feat_v, rows_v, in_sems, g_sems, out_sems):
    cid = lax.axis_index("c")
    sid = lax.axis_index("s")
    wid = cid * NS + sid
    base = SPLIT + wid * RPW
    obase = wid * RPW

    pltpu.sync_copy(idx_hbm.at[wid], idx_v)

    def start_load(r, b):
        pltpu.async_copy(feat_hbm.at[pl.ds(base + r * CHUNK, CHUNK)],
                         feat_v.at[b], in_sems.at[b])
        pltpu.async_copy(table_hbm.at[idx_v.at[r]],
                         rows_v.at[b], g_sems.at[b])

    def wait(sem_arr, b, src, dst):
        pltpu.make_async_copy(src, dst, sem_arr.at[b]).wait()

    for b in range(NBUF):
        start_load(b, b)

    def group(g, carry):
        for b in range(NBUF):
            r = g * NBUF + b
            wait(in_sems, b, feat_hbm.at[pl.ds(base, CHUNK)], feat_v.at[b])
            wait(g_sems, b, table_hbm.at[idx_v.at[0]], rows_v.at[b])
            fb = feat_v.at[b]
            rb = rows_v.at[b]

            def add_row(k, c):
                for j in range(HIDDEN // LANES):
                    sl = pl.ds(j * LANES, LANES)
                    plsc.addupdate(fb.at[k, sl], rb[k, sl])
                return c

            lax.fori_loop(0, CHUNK, add_row, 0)
            pltpu.async_copy(fb,
                             out_hbm.at[pl.ds(obase + r * CHUNK, CHUNK)],
                             out_sems.at[b])
        for b in range(NBUF):
            r_next = (g + 1) * NBUF + b

            @pl.when(r_next < ROUNDS)
            def _():
                wait(out_sems, b, feat_v.at[b],
                     out_hbm.at[pl.ds(obase, CHUNK)])
                start_load(r_next, b)
        return carry

    lax.fori_loop(0, GROUPS, group, 0)
    for b in range(NBUF):
        wait(out_sems, b, feat_v.at[b], out_hbm.at[pl.ds(obase, CHUNK)])


def _sc_part(features, idx, temporal_embedding):
    idx3 = idx[SPLIT:].reshape(NW, ROUNDS, CHUNK)
    mesh = plsc.VectorSubcoreMesh(core_axis_name="c", subcore_axis_name="s")
    run = pl.kernel(
        _sc_body,
        out_type=jax.ShapeDtypeStruct((SC_ROWS, HIDDEN), features.dtype),
        mesh=mesh,
        scratch_types=[
            pltpu.VMEM((ROUNDS, CHUNK), jnp.int32),
            pltpu.VMEM((NBUF, CHUNK, HIDDEN), jnp.float32),
            pltpu.VMEM((NBUF, CHUNK, HIDDEN), jnp.float32),
            pltpu.SemaphoreType.DMA((NBUF,)),
            pltpu.SemaphoreType.DMA((NBUF,)),
            pltpu.SemaphoreType.DMA((NBUF,)),
        ],
    )
    return run(features, idx3, temporal_embedding)


@jax.jit
def kernel(features, temporal_embedding):
    n = features.shape[0]
    temporal_pos = jnp.linspace(0.0, 1.0, n, dtype=features.dtype)
    idx = jnp.clip(temporal_pos * TABLE_ROWS, 0, TABLE_ROWS - 1).astype(jnp.int32)
    tc_out = _tc_part(features, idx, temporal_embedding)
    sc_out = _sc_part(features, idx, temporal_embedding)
    return jnp.concatenate([tc_out, sc_out], axis=0)


# speedup vs baseline: 2.2291x; 2.2291x over previous
import functools

import jax
import jax.numpy as jnp
from jax.experimental import pallas as pl
from jax.experimental.pallas import tpu as pltpu

N_ROWS = 16384
HIDDEN = 1024
TABLE_ROWS = 1000

B = 1024
NSTEPS = N_ROWS // B
NBUF = 4
LA = NBUF - 1
HALF = B // 2
SUB = 16


def _body(idx_smem, feat_any, ivec_any, table_any, out_any,
          tbl_v, fin, ivin, fout, fsem, isem, osem, tsem):
    s = pl.program_id(0)

    def issue_in(step):
        b = step % NBUF
        for h in range(2):
            pltpu.async_copy(
                feat_any.at[pl.ds(step * B + h * HALF, HALF)],
                fin.at[b, pl.ds(h * HALF, HALF)], fsem.at[b, h])
        pltpu.async_copy(ivec_any.at[pl.ds(step * B, B)],
                         ivin.at[b], isem.at[b])

    @pl.when(s == 0)
    def _():
        pltpu.async_copy(table_any, tbl_v, tsem).wait()
        for st in range(LA):
            issue_in(st)

    @pl.when(s + LA < NSTEPS)
    def _():
        issue_in(s + LA)

    b = s % NBUF

    @pl.when(s >= NBUF)
    def _():
        for h in range(2):
            pltpu.make_async_copy(
                fout.at[b, pl.ds(h * HALF, HALF)],
                out_any.at[pl.ds(h * HALF, HALF)], osem.at[b, h]).wait()

    for h in range(2):
        pltpu.make_async_copy(
            feat_any.at[pl.ds(h * HALF, HALF)],
            fin.at[b, pl.ds(h * HALF, HALF)], fsem.at[b, h]).wait()
    pltpu.make_async_copy(ivec_any.at[pl.ds(0, B)], ivin.at[b],
                          isem.at[b]).wait()

    for k in range(B // SUB):
        base = s * B + k * SUB
        r0 = idx_smem[base]
        r1 = idx_smem[base + SUB - 1]
        a = tbl_v[pl.ds(r0, 1), :]
        c = tbl_v[pl.ds(r1, 1), :]
        idx_v = ivin[b, pl.ds(k * SUB, SUB), :]
        mask = idx_v == r0
        sl = pl.ds(k * SUB, SUB)
        fout[b, sl, :] = fin[b, sl, :] + jnp.where(mask, a, c)

    for h in range(2):
        pltpu.async_copy(
            fout.at[b, pl.ds(h * HALF, HALF)],
            out_any.at[pl.ds(s * B + h * HALF, HALF)], osem.at[b, h])

    @pl.when(s == NSTEPS - 1)
    def _():
        for d in range(NBUF):
            bb = (s - d) % NBUF
            for h in range(2):
                pltpu.make_async_copy(
                    fout.at[bb, pl.ds(h * HALF, HALF)],
                    out_any.at[pl.ds(h * HALF, HALF)], osem.at[bb, h]).wait()


@jax.jit
def kernel(features, temporal_embedding):
    n = features.shape[0]
    temporal_pos = jnp.linspace(0.0, 1.0, n, dtype=features.dtype)
    idx = jnp.clip(temporal_pos * TABLE_ROWS, 0, TABLE_ROWS - 1).astype(jnp.int32)
    idx_vec = idx.reshape(n, 1)

    grid_spec = pltpu.PrefetchScalarGridSpec(
        num_scalar_prefetch=1,
        grid=(NSTEPS,),
        in_specs=[
            pl.BlockSpec(memory_space=pl.ANY),
            pl.BlockSpec(memory_space=pl.ANY),
            pl.BlockSpec(memory_space=pl.ANY),
        ],
        out_specs=pl.BlockSpec(memory_space=pl.ANY),
        scratch_shapes=[
            pltpu.VMEM((TABLE_ROWS, HIDDEN), jnp.float32),
            pltpu.VMEM((NBUF, B, HIDDEN), jnp.float32),
            pltpu.VMEM((NBUF, B, 1), jnp.int32),
            pltpu.VMEM((NBUF, B, HIDDEN), jnp.float32),
            pltpu.SemaphoreType.DMA((NBUF, 2)),
            pltpu.SemaphoreType.DMA((NBUF,)),
            pltpu.SemaphoreType.DMA((NBUF, 2)),
            pltpu.SemaphoreType.DMA,
        ],
    )
    return pl.pallas_call(
        _body,
        grid_spec=grid_spec,
        out_shape=jax.ShapeDtypeStruct((n, HIDDEN), features.dtype),
        compiler_params=pltpu.CompilerParams(
            dimension_semantics=("arbitrary",)),
    )(idx, features, idx_vec, temporal_embedding)
